# Initial kernel scaffold; baseline (speedup 1.0000x reference)
import jax
import jax.numpy as jnp
from jax.experimental import pallas as pl


def kernel(h_user, h_item, params, edge_click, edge_rev):
    raise NotImplementedError("write your pallas kernel here")


# speedup vs baseline: 7.4320x; 7.4320x over previous
import functools
import math

import jax
import jax.numpy as jnp
from jax import lax
from jax.experimental import pallas as pl
from jax.experimental.pallas import tpu as pltpu
from jax.experimental.pallas import tpu_sc as plsc

N = 10000
E_EDGES = 160000
D = 256
NH = 8
DK = 32
SQRT_DK = math.sqrt(DK)
NC = 1
NS = 16
NGRP = 2
HPG = NH // NGRP
GCOLS = D // NGRP
DCOLS = 16
CHUNK = 32
CWORDS = CHUNK // 2
EPADT = 10016
NPAD = 10016
MROWS = 10240
GARBAGE = 10008
ZCH = 40
DPACK = MROWS + MROWS // 8


def _proj_body(pri_ref, h_ref, wq_ref, bq_ref, wk_ref, bk_ref, wv_ref, bv_ref,
               ratt_ref, rmsg_ref, q_ref, k_ref, v_ref):
    x = h_ref[...]
    q_ref[...] = jnp.dot(x, wq_ref[...], preferred_element_type=jnp.float32) + bq_ref[...]
    wk = wk_ref[...]
    wv = wv_ref[...]
    bk = bk_ref[...]
    bv = bv_ref[...]
    wkf_parts = []
    wvf_parts = []
    bkf_parts = []
    bvf_parts = []
    for h in range(NH):
        sl = slice(h * DK, (h + 1) * DK)
        scale = pri_ref[0, h] / SQRT_DK
        ra = ratt_ref[h]
        rm = rmsg_ref[h]
        wkf_parts.append(jnp.dot(wk[:, sl], ra, preferred_element_type=jnp.float32) * scale)
        wvf_parts.append(jnp.dot(wv[:, sl], rm, preferred_element_type=jnp.float32))
        bkf_parts.append(jnp.dot(bk[:, sl], ra, preferred_element_type=jnp.float32) * scale)
        bvf_parts.append(jnp.dot(bv[:, sl], rm, preferred_element_type=jnp.float32))
    wkf = jnp.concatenate(wkf_parts, axis=1)
    wvf = jnp.concatenate(wvf_parts, axis=1)
    bkf = jnp.concatenate(bkf_parts, axis=1)
    bvf = jnp.concatenate(bvf_parts, axis=1)
    k_ref[...] = jnp.dot(x, wkf, preferred_element_type=jnp.float32) + bkf
    v_ref[...] = jnp.dot(x, wvf, preferred_element_type=jnp.float32) + bvf


def _projections(h, wq, bq, wk, bk, wv, bv, ratt, rmsg, rpri):
    br = 2000
    grid = (N // br,)
    out_shape = [jax.ShapeDtypeStruct((N, D), jnp.float32)] * 3
    row_spec = pl.BlockSpec((br, D), lambda i: (i, 0))
    full_spec = pl.BlockSpec((D, D), lambda i: (0, 0))
    bias_spec = pl.BlockSpec((1, D), lambda i: (0, 0))
    r_spec = pl.BlockSpec((NH, DK, DK), lambda i: (0, 0, 0))
    return pl.pallas_call(
        _proj_body,
        grid=grid,
        in_specs=[
            pl.BlockSpec(memory_space=pltpu.SMEM),
            row_spec,
            full_spec, bias_spec,
            full_spec, bias_spec,
            full_spec, bias_spec,
            r_spec, r_spec,
        ],
        out_specs=[row_spec, row_spec, row_spec],
        out_shape=out_shape,
    )(rpri.reshape(1, NH), h, wq, bq.reshape(1, D), wk, bk.reshape(1, D),
      wv, bv.reshape(1, D), ratt, rmsg)


def _edge_body(q_hbm, k_hbm, v_hbm, src_hbm, dst_hbm, zrows_hbm, m_out,
               sidx, didx, asidx, adidx, rdidx, dnidx,
               qrows, krows, vrows, pden, m_sh):
    s = lax.axis_index("s")
    lanes = lax.broadcasted_iota(jnp.int32, (16,), 0)
    zvec = jnp.zeros((16,), jnp.float32)
    rbase = s * (DPACK // NS)

    for ph in range(2 * NGRP):
        rel, t = divmod(ph, NGRP)
        coff = ph * NPAD
        ebase = rel * (EPADT * NS // 2)

        for k in range(DPACK // NS // ZCH):
            pltpu.sync_copy(zrows_hbm, m_sh.at[pl.ds(rbase + k * ZCH, ZCH)])

        plsc.subcore_barrier()

        def _chunk_body(ci, _):
            off = ebase + s * (EPADT // 2) + ci * CWORDS
            pltpu.sync_copy(src_hbm.at[pl.ds(off, CWORDS)], sidx)
            pltpu.sync_copy(dst_hbm.at[pl.ds(off, CWORDS)], didx)
            for j in range(CHUNK // 32):
                ws = sidx[pl.ds(j * 16, 16)]
                wd = didx[pl.ds(j * 16, 16)]
                lo_s = ws & 0xFFFF
                hi_s = lax.shift_right_logical(ws, 16)
                lo_d = wd & 0xFFFF
                hi_d = lax.shift_right_logical(wd, 16)
                asidx[pl.ds(j * 32, 16)] = lo_s + coff
                asidx[pl.ds(j * 32 + 16, 16)] = hi_s + coff
                adidx[pl.ds(j * 32, 16)] = lo_d + coff
                adidx[pl.ds(j * 32 + 16, 16)] = hi_d + coff
                rdidx[pl.ds(j * 32, 16)] = lo_d
                rdidx[pl.ds(j * 32 + 16, 16)] = hi_d
                dnidx[pl.ds(j * 32, 16)] = (
                    lax.shift_right_logical(lo_d, 3) + MROWS)
                dnidx[pl.ds(j * 32 + 16, 16)] = (
                    lax.shift_right_logical(hi_d, 3) + MROWS)
            pltpu.sync_copy(q_hbm.at[adidx], qrows)
            pltpu.sync_copy(k_hbm.at[asidx], krows)
            pltpu.sync_copy(v_hbm.at[asidx], vrows)

            def _group_g(g, _):
                dv = rdidx[pl.ds(g * 16, 16)]
                for i in range(16):
                    e = g * 16 + i
                    blk = dv[i] & 7
                    prow = zvec
                    for h in range(HPG):
                        b = h * DK
                        a = (qrows[e, pl.ds(b, 16)] * krows[e, pl.ds(b, 16)]
                             + qrows[e, pl.ds(b + 16, 16)]
                             * krows[e, pl.ds(b + 16, 16)])
                        for sh in (1, 2, 4, 8):
                            a = a + a[lanes ^ sh]
                        pv = jnp.exp(a)
                        prow = jnp.where(lanes == h + HPG * t, pv, prow)
                        vrows[e, pl.ds(b, 16)] = vrows[e, pl.ds(b, 16)] * pv
                        vrows[e, pl.ds(b + 16, 16)] = (
                            vrows[e, pl.ds(b + 16, 16)] * pv)
                    for bb in range(8):
                        pden[e, pl.ds(bb * 16, 16)] = jnp.where(
                            blk == bb, prow, zvec)
                return 0

            lax.fori_loop(0, CHUNK // 16, _group_g, 0)

            pltpu.sync_copy(vrows, m_sh.at[rdidx], add=True)
            pltpu.sync_copy(pden, m_sh.at[dnidx], add=True)
            return 0

        lax.fori_loop(0, EPADT // CHUNK, _chunk_body, 0)

        plsc.subcore_barrier()

        for k in range(DPACK // NS // ZCH):
            r0 = rbase + k * ZCH
            pltpu.sync_copy(m_sh.at[pl.ds(r0, ZCH)],
                            m_out.at[pl.ds(ph * DPACK + r0, ZCH)])

        plsc.subcore_barrier()


def _make_edge_kernel():
    mesh = plsc.VectorSubcoreMesh(core_axis_name="c", subcore_axis_name="s",
                                  num_cores=NC, num_subcores=NS)
    return pl.kernel(
        _edge_body,
        out_type=jax.ShapeDtypeStruct((2 * NGRP * DPACK, GCOLS), jnp.float32),
        mesh=mesh,
        scratch_types=[
            pltpu.VMEM((CWORDS,), jnp.int32),
            pltpu.VMEM((CWORDS,), jnp.int32),
            pltpu.VMEM((CHUNK,), jnp.int32),
            pltpu.VMEM((CHUNK,), jnp.int32),
            pltpu.VMEM((CHUNK,), jnp.int32),
            pltpu.VMEM((CHUNK,), jnp.int32),
            pltpu.VMEM((CHUNK, GCOLS), jnp.float32),
            pltpu.VMEM((CHUNK, GCOLS), jnp.float32),
            pltpu.VMEM((CHUNK, GCOLS), jnp.float32),
            pltpu.VMEM((CHUNK, GCOLS), jnp.float32),
            pltpu.VMEM_SHARED((DPACK, GCOLS), jnp.float32),
        ],
    )


def _out_body(alpha_ref, m_ref, den_ref, h_ref, wa_ref, ba_ref, out_ref):
    m = m_ref[...]
    den = den_ref[...]
    parts = []
    for h in range(NH):
        sl = slice(h * DK, (h + 1) * DK)
        parts.append(m[:, sl] / (den[:, h:h + 1] + 1e-9))
    agg = jnp.concatenate(parts, axis=1)
    alpha = alpha_ref[0, 0]
    out_ref[...] = ((jnp.dot(agg, wa_ref[...], preferred_element_type=jnp.float32)
                     + ba_ref[...]) * alpha + h_ref[...] * (1.0 - alpha))


def _output(m, den, h, wa, ba, alpha):
    br = 2000
    grid = (N // br,)
    row_spec = pl.BlockSpec((br, D), lambda i: (i, 0))
    return pl.pallas_call(
        _out_body,
        grid=grid,
        in_specs=[
            pl.BlockSpec(memory_space=pltpu.SMEM),
            row_spec,
            pl.BlockSpec((br, NH), lambda i: (i, 0)),
            row_spec,
            pl.BlockSpec((D, D), lambda i: (0, 0)),
            pl.BlockSpec((1, D), lambda i: (0, 0)),
        ],
        out_specs=row_spec,
        out_shape=jax.ShapeDtypeStruct((N, D), jnp.float32),
    )(alpha.reshape(1, 1), m, den, h, wa, ba.reshape(1, D))


def kernel(h_user, h_item, params, edge_click, edge_rev):
    p = params
    q_u, khat_u, vhat_u = _projections(
        h_user, p['Wq_user'], p['bq_user'], p['Wk_user'], p['bk_user'],
        p['Wv_user'], p['bv_user'], p['relation_att'][0], p['relation_msg'][0],
        p['relation_pri'][0])
    q_i, khat_i, vhat_i = _projections(
        h_item, p['Wq_item'], p['bq_item'], p['Wk_item'], p['bk_item'],
        p['Wv_item'], p['bv_item'], p['relation_att'][1], p['relation_msg'][1],
        p['relation_pri'][1])

    tpad = jnp.zeros((NPAD - N, GCOLS), jnp.float32)

    def stack_cols(xs):
        return jnp.concatenate(
            [y for x in xs for g in range(NGRP)
             for y in (x[:, g * GCOLS:(g + 1) * GCOLS], tpad)],
            axis=0)

    edge_kernel = _make_edge_kernel()
    ec = edge_click.astype(jnp.int32)
    er = edge_rev.astype(jnp.int32)

    def pack_ids(a, pad_id):
        a2 = a.reshape(NS, E_EDGES // NS)
        padv = jnp.full((NS, EPADT - E_EDGES // NS), pad_id, jnp.int32)
        a2 = jnp.concatenate([a2, padv], axis=1).reshape(-1)
        return a2[0::2] | (a2[1::2] << 16)

    m_flat = edge_kernel(
        stack_cols([q_i, q_u]),
        stack_cols([khat_u, khat_i]),
        stack_cols([vhat_u, vhat_i]),
        jnp.concatenate([pack_ids(ec[0], 0), pack_ids(er[0], 0)]),
        jnp.concatenate([pack_ids(ec[1], GARBAGE), pack_ids(er[1], GARBAGE)]),
        jnp.zeros((ZCH, GCOLS), jnp.float32))
    m_all = m_flat.reshape(2 * NGRP, DPACK, GCOLS)

    def unstack(rel):
        msum = jnp.concatenate(
            [m_all[rel * NGRP + g][:N] for g in range(NGRP)], axis=1)

        def den_half(g):
            rows = m_all[rel * NGRP + g][MROWS:DPACK]
            return rows.reshape(MROWS, DCOLS)[:N, g * HPG:(g + 1) * HPG]

        den = jnp.concatenate([den_half(0), den_half(1)], axis=1)
        return msum, den

    m_item, den_item = unstack(0)
    m_user, den_user = unstack(1)

    alpha_u = jax.nn.sigmoid(p['skip'][0])
    alpha_i = jax.nn.sigmoid(p['skip'][1])
    out_u = _output(m_user, den_user, h_user, p['Wa_user'], p['ba_user'], alpha_u)
    out_i = _output(m_item, den_item, h_item, p['Wa_item'], p['ba_item'], alpha_i)
    return jnp.stack([out_u, out_i])


# speedup vs baseline: 14.1607x; 1.9054x over previous
import functools
import math

import jax
import jax.numpy as jnp
from jax import lax
from jax.experimental import pallas as pl
from jax.experimental.pallas import tpu as pltpu
from jax.experimental.pallas import tpu_sc as plsc

N = 10000
E_EDGES = 160000
D = 256
NH = 8
DK = 32
SQRT_DK = math.sqrt(DK)
NC = 2
NS = 16
NGRP = 2
HPG = NH // NGRP
GCOLS = D // NGRP
DCOLS = 16
CHUNK = 32
CWORDS = CHUNK // 2
EPADT = 10016
NPAD = 10016
MROWS = 10240
GARBAGE = 10008
ZCH = 40
DPACK = MROWS + MROWS // 8


def _proj_body(pri_ref, h_ref, wq_ref, bq_ref, wk_ref, bk_ref, wv_ref, bv_ref,
               ratt_ref, rmsg_ref, q_ref, k_ref, v_ref):
    x = h_ref[...]
    q_ref[...] = jnp.dot(x, wq_ref[...], preferred_element_type=jnp.float32) + bq_ref[...]
    wk = wk_ref[...]
    wv = wv_ref[...]
    bk = bk_ref[...]
    bv = bv_ref[...]
    wkf_parts = []
    wvf_parts = []
    bkf_parts = []
    bvf_parts = []
    for h in range(NH):
        sl = slice(h * DK, (h + 1) * DK)
        scale = pri_ref[0, h] / SQRT_DK
        ra = ratt_ref[h]
        rm = rmsg_ref[h]
        wkf_parts.append(jnp.dot(wk[:, sl], ra, preferred_element_type=jnp.float32) * scale)
        wvf_parts.append(jnp.dot(wv[:, sl], rm, preferred_element_type=jnp.float32))
        bkf_parts.append(jnp.dot(bk[:, sl], ra, preferred_element_type=jnp.float32) * scale)
        bvf_parts.append(jnp.dot(bv[:, sl], rm, preferred_element_type=jnp.float32))
    wkf = jnp.concatenate(wkf_parts, axis=1)
    wvf = jnp.concatenate(wvf_parts, axis=1)
    bkf = jnp.concatenate(bkf_parts, axis=1)
    bvf = jnp.concatenate(bvf_parts, axis=1)
    k_ref[...] = jnp.dot(x, wkf, preferred_element_type=jnp.float32) + bkf
    v_ref[...] = jnp.dot(x, wvf, preferred_element_type=jnp.float32) + bvf


def _projections(h, wq, bq, wk, bk, wv, bv, ratt, rmsg, rpri):
    br = 2000
    grid = (N // br,)
    out_shape = [jax.ShapeDtypeStruct((N, D), jnp.float32)] * 3
    row_spec = pl.BlockSpec((br, D), lambda i: (i, 0))
    full_spec = pl.BlockSpec((D, D), lambda i: (0, 0))
    bias_spec = pl.BlockSpec((1, D), lambda i: (0, 0))
    r_spec = pl.BlockSpec((NH, DK, DK), lambda i: (0, 0, 0))
    return pl.pallas_call(
        _proj_body,
        grid=grid,
        in_specs=[
            pl.BlockSpec(memory_space=pltpu.SMEM),
            row_spec,
            full_spec, bias_spec,
            full_spec, bias_spec,
            full_spec, bias_spec,
            r_spec, r_spec,
        ],
        out_specs=[row_spec, row_spec, row_spec],
        out_shape=out_shape,
    )(rpri.reshape(1, NH), h, wq, bq.reshape(1, D), wk, bk.reshape(1, D),
      wv, bv.reshape(1, D), ratt, rmsg)


def _edge_body(q_hbm, k_hbm, v_hbm, src_hbm, dst_hbm, zrows_hbm, m_out,
               sidx, didx, asidx, adidx, rdidx, dnidx,
               qrows, krows, vrows, pden, m_sh):
    c = lax.axis_index("c")
    s = lax.axis_index("s")
    lanes = lax.broadcasted_iota(jnp.int32, (16,), 0)
    zvec = jnp.zeros((16,), jnp.float32)
    rbase = s * (DPACK // NS)

    for pp in range(2 * NGRP // NC):
        ph = NC * pp + c
        rel = ph // NGRP
        t = ph % NGRP
        coff = ph * NPAD
        ebase = rel * (EPADT * NS // 2)

        for k in range(DPACK // NS // ZCH):
            pltpu.sync_copy(zrows_hbm, m_sh.at[pl.ds(rbase + k * ZCH, ZCH)])

        plsc.subcore_barrier()

        def _chunk_body(ci, _):
            off = ebase + s * (EPADT // 2) + ci * CWORDS
            pltpu.sync_copy(src_hbm.at[pl.ds(off, CWORDS)], sidx)
            pltpu.sync_copy(dst_hbm.at[pl.ds(off, CWORDS)], didx)
            for j in range(CHUNK // 32):
                ws = sidx[pl.ds(j * 16, 16)]
                wd = didx[pl.ds(j * 16, 16)]
                lo_s = ws & 0xFFFF
                hi_s = lax.shift_right_logical(ws, 16)
                lo_d = wd & 0xFFFF
                hi_d = lax.shift_right_logical(wd, 16)
                asidx[pl.ds(j * 32, 16)] = lo_s + coff
                asidx[pl.ds(j * 32 + 16, 16)] = hi_s + coff
                adidx[pl.ds(j * 32, 16)] = lo_d + coff
                adidx[pl.ds(j * 32 + 16, 16)] = hi_d + coff
                rdidx[pl.ds(j * 32, 16)] = lo_d
                rdidx[pl.ds(j * 32 + 16, 16)] = hi_d
                dnidx[pl.ds(j * 32, 16)] = (
                    lax.shift_right_logical(lo_d, 3) + MROWS)
                dnidx[pl.ds(j * 32 + 16, 16)] = (
                    lax.shift_right_logical(hi_d, 3) + MROWS)
            pltpu.sync_copy(q_hbm.at[adidx], qrows)
            pltpu.sync_copy(k_hbm.at[asidx], krows)
            pltpu.sync_copy(v_hbm.at[asidx], vrows)

            def _group_g(g, _):
                dv = rdidx[pl.ds(g * 16, 16)]
                for i in range(16):
                    e = g * 16 + i
                    blk = dv[i] & 7
                    prow = zvec
                    for h in range(HPG):
                        b = h * DK
                        a = (qrows[e, pl.ds(b, 16)] * krows[e, pl.ds(b, 16)]
                             + qrows[e, pl.ds(b + 16, 16)]
                             * krows[e, pl.ds(b + 16, 16)])
                        for sh in (1, 2, 4, 8):
                            a = a + a[lanes ^ sh]
                        pv = jnp.exp(a)
                        prow = jnp.where(lanes == h + HPG * t, pv, prow)
                        vrows[e, pl.ds(b, 16)] = vrows[e, pl.ds(b, 16)] * pv
                        vrows[e, pl.ds(b + 16, 16)] = (
                            vrows[e, pl.ds(b + 16, 16)] * pv)
                    for bb in range(8):
                        pden[e, pl.ds(bb * 16, 16)] = jnp.where(
                            blk == bb, prow, zvec)
                return 0

            lax.fori_loop(0, CHUNK // 16, _group_g, 0)

            pltpu.sync_copy(vrows, m_sh.at[rdidx], add=True)
            pltpu.sync_copy(pden, m_sh.at[dnidx], add=True)
            return 0

        lax.fori_loop(0, EPADT // CHUNK, _chunk_body, 0)

        plsc.subcore_barrier()

        for k in range(DPACK // NS // ZCH):
            r0 = rbase + k * ZCH
            pltpu.sync_copy(m_sh.at[pl.ds(r0, ZCH)],
                            m_out.at[pl.ds(ph * DPACK + r0, ZCH)])

        plsc.subcore_barrier()


def _make_edge_kernel():
    mesh = plsc.VectorSubcoreMesh(core_axis_name="c", subcore_axis_name="s",
                                  num_cores=NC, num_subcores=NS)
    return pl.kernel(
        _edge_body,
        out_type=jax.ShapeDtypeStruct((2 * NGRP * DPACK, GCOLS), jnp.float32),
        mesh=mesh,
        scratch_types=[
            pltpu.VMEM((CWORDS,), jnp.int32),
            pltpu.VMEM((CWORDS,), jnp.int32),
            pltpu.VMEM((CHUNK,), jnp.int32),
            pltpu.VMEM((CHUNK,), jnp.int32),
            pltpu.VMEM((CHUNK,), jnp.int32),
            pltpu.VMEM((CHUNK,), jnp.int32),
            pltpu.VMEM((CHUNK, GCOLS), jnp.float32),
            pltpu.VMEM((CHUNK, GCOLS), jnp.float32),
            pltpu.VMEM((CHUNK, GCOLS), jnp.float32),
            pltpu.VMEM((CHUNK, GCOLS), jnp.float32),
            pltpu.VMEM_SHARED((DPACK, GCOLS), jnp.float32),
        ],
    )


def _out_body(alpha_ref, m_ref, den_ref, h_ref, wa_ref, ba_ref, out_ref):
    m = m_ref[...]
    den = den_ref[...]
    parts = []
    for h in range(NH):
        sl = slice(h * DK, (h + 1) * DK)
        parts.append(m[:, sl] / (den[:, h:h + 1] + 1e-9))
    agg = jnp.concatenate(parts, axis=1)
    alpha = alpha_ref[0, 0]
    out_ref[...] = ((jnp.dot(agg, wa_ref[...], preferred_element_type=jnp.float32)
                     + ba_ref[...]) * alpha + h_ref[...] * (1.0 - alpha))


def _output(m, den, h, wa, ba, alpha):
    br = 2000
    grid = (N // br,)
    row_spec = pl.BlockSpec((br, D), lambda i: (i, 0))
    return pl.pallas_call(
        _out_body,
        grid=grid,
        in_specs=[
            pl.BlockSpec(memory_space=pltpu.SMEM),
            row_spec,
            pl.BlockSpec((br, NH), lambda i: (i, 0)),
            row_spec,
            pl.BlockSpec((D, D), lambda i: (0, 0)),
            pl.BlockSpec((1, D), lambda i: (0, 0)),
        ],
        out_specs=row_spec,
        out_shape=jax.ShapeDtypeStruct((N, D), jnp.float32),
    )(alpha.reshape(1, 1), m, den, h, wa, ba.reshape(1, D))


def kernel(h_user, h_item, params, edge_click, edge_rev):
    p = params
    q_u, khat_u, vhat_u = _projections(
        h_user, p['Wq_user'], p['bq_user'], p['Wk_user'], p['bk_user'],
        p['Wv_user'], p['bv_user'], p['relation_att'][0], p['relation_msg'][0],
        p['relation_pri'][0])
    q_i, khat_i, vhat_i = _projections(
        h_item, p['Wq_item'], p['bq_item'], p['Wk_item'], p['bk_item'],
        p['Wv_item'], p['bv_item'], p['relation_att'][1], p['relation_msg'][1],
        p['relation_pri'][1])

    tpad = jnp.zeros((NPAD - N, GCOLS), jnp.float32)

    def stack_cols(xs):
        return jnp.concatenate(
            [y for x in xs for g in range(NGRP)
             for y in (x[:, g * GCOLS:(g + 1) * GCOLS], tpad)],
            axis=0)

    edge_kernel = _make_edge_kernel()
    ec = edge_click.astype(jnp.int32)
    er = edge_rev.astype(jnp.int32)

    def pack_ids(a, pad_id):
        a2 = a.reshape(NS, E_EDGES // NS)
        padv = jnp.full((NS, EPADT - E_EDGES // NS), pad_id, jnp.int32)
        a2 = jnp.concatenate([a2, padv], axis=1).reshape(-1)
        return a2[0::2] | (a2[1::2] << 16)

    m_flat = edge_kernel(
        stack_cols([q_i, q_u]),
        stack_cols([khat_u, khat_i]),
        stack_cols([vhat_u, vhat_i]),
        jnp.concatenate([pack_ids(ec[0], 0), pack_ids(er[0], 0)]),
        jnp.concatenate([pack_ids(ec[1], GARBAGE), pack_ids(er[1], GARBAGE)]),
        jnp.zeros((ZCH, GCOLS), jnp.float32))
    m_all = m_flat.reshape(2 * NGRP, DPACK, GCOLS)

    def unstack(rel):
        msum = jnp.concatenate(
            [m_all[rel * NGRP + g][:N] for g in range(NGRP)], axis=1)

        def den_half(g):
            rows = m_all[rel * NGRP + g][MROWS:DPACK]
            return rows.reshape(MROWS, DCOLS)[:N, g * HPG:(g + 1) * HPG]

        den = jnp.concatenate([den_half(0), den_half(1)], axis=1)
        return msum, den

    m_item, den_item = unstack(0)
    m_user, den_user = unstack(1)

    alpha_u = jax.nn.sigmoid(p['skip'][0])
    alpha_i = jax.nn.sigmoid(p['skip'][1])
    out_u = _output(m_user, den_user, h_user, p['Wa_user'], p['ba_user'], alpha_u)
    out_i = _output(m_item, den_item, h_item, p['Wa_item'], p['ba_item'], alpha_i)
    return jnp.stack([out_u, out_i])
